# Initial kernel scaffold; baseline (speedup 1.0000x reference)
import jax
import jax.numpy as jnp
from jax.experimental import pallas as pl


def kernel(x, idx, dis, angle_t):
    raise NotImplementedError("write your pallas kernel here")


# speedup vs baseline: 5.0771x; 5.0771x over previous
import jax
import jax.numpy as jnp
from jax import lax
from jax.experimental import pallas as pl
from jax.experimental.pallas import tpu as pltpu
from jax.experimental.pallas import tpu_sc as plsc

H, W, D = 16384, 20, 64
S = 2 * W
OUT_W = W * D + S
NC, NS = 2, 16
NW = NC * NS
RPW = H // NW
C = 64
NCHUNK = RPW // C


def _body(x_hbm, idxt_hbm, da_hbm, out_hbm, idx_v, rows_v, da_v, sem):
  wid = lax.axis_index("s") * NC + lax.axis_index("c")
  pltpu.sync_copy(idxt_hbm.at[:, pl.ds(wid * RPW, RPW)], idx_v)

  def chunk(c, carry):
    base_row = wid * RPW + c * C
    for j in range(W):
      pltpu.async_copy(
          x_hbm.at[idx_v.at[j, pl.ds(c * C, C)]], rows_v.at[j], sem)
    pltpu.sync_copy(da_hbm.at[pl.ds(base_row, C), :], da_v)
    pltpu.sync_copy(da_v, out_hbm.at[pl.ds(base_row, C), pl.ds(W * D, S)])
    for j in range(W):
      pltpu.make_async_copy(
          x_hbm.at[idx_v.at[j, pl.ds(c * C, C)]], rows_v.at[j], sem).wait()
      pltpu.sync_copy(rows_v.at[j],
                      out_hbm.at[pl.ds(base_row, C), pl.ds(j * D, D)])
    return carry

  lax.fori_loop(0, NCHUNK, chunk, 0)


@jax.jit
def _run(x, idxt, da):
  mesh = plsc.VectorSubcoreMesh(core_axis_name="c", subcore_axis_name="s")
  return pl.kernel(
      _body,
      out_type=jax.ShapeDtypeStruct((H, OUT_W), jnp.float32),
      mesh=mesh,
      scratch_types=[
          pltpu.VMEM((W, RPW), jnp.int32),
          pltpu.VMEM((W, C, D), jnp.float32),
          pltpu.VMEM((C, S), jnp.float32),
          pltpu.SemaphoreType.DMA,
      ],
      compiler_params=pltpu.CompilerParams(use_tc_tiling_on_sc=False),
  )(x, idxt, da)


def kernel(x, idx, dis, angle_t):
  idxt = idx.astype(jnp.int32).T
  da = jnp.concatenate([dis, angle_t], axis=1)
  return _run(x, idxt, da)


# speedup vs baseline: 5.0998x; 1.0045x over previous
import jax
import jax.numpy as jnp
from jax import lax
from jax.experimental import pallas as pl
from jax.experimental.pallas import tpu as pltpu
from jax.experimental.pallas import tpu_sc as plsc

H, W, D = 16384, 20, 64
S = 2 * W
OUT_W = W * D + S
NC, NS = 2, 16
NW = NC * NS
RPW = H // NW
NPAIR = W // 2


def _body(x_hbm, idxt_hbm, da_hbm, out_hbm, idx_v, rows0, rows1, da_v,
          sg0, sg1):
  wid = lax.axis_index("s") * NC + lax.axis_index("c")
  wbase = wid * RPW
  bufs = (rows0, rows1)
  sems = (sg0, sg1)
  pltpu.sync_copy(idxt_hbm.at[:, pl.ds(wbase, RPW)], idx_v)

  def gather(j, b):
    pltpu.async_copy(x_hbm.at[idx_v.at[j]], bufs[b], sems[b])

  gather(0, 0)
  gather(1, 1)
  pltpu.sync_copy(da_hbm.at[pl.ds(wbase, RPW), :], da_v)
  pltpu.sync_copy(da_v, out_hbm.at[pl.ds(wbase, RPW), pl.ds(W * D, S)])

  def pair(g, carry):
    for b in (0, 1):
      j = 2 * g + b
      pltpu.make_async_copy(x_hbm.at[idx_v.at[j]], bufs[b], sems[b]).wait()
      col = pl.multiple_of(j * D, D)
      pltpu.sync_copy(bufs[b],
                      out_hbm.at[pl.ds(wbase, RPW), pl.ds(col, D)])

      @pl.when(g < NPAIR - 1)
      def _():
        gather(j + 2, b)

    return carry

  lax.fori_loop(0, NPAIR, pair, 0)


@jax.jit
def _run(x, idxt, da):
  mesh = plsc.VectorSubcoreMesh(core_axis_name="c", subcore_axis_name="s")
  return pl.kernel(
      _body,
      out_type=jax.ShapeDtypeStruct((H, OUT_W), jnp.float32),
      mesh=mesh,
      scratch_types=[
          pltpu.VMEM((W, RPW), jnp.int32),
          pltpu.VMEM((RPW, D), jnp.float32),
          pltpu.VMEM((RPW, D), jnp.float32),
          pltpu.VMEM((RPW, S), jnp.float32),
          pltpu.SemaphoreType.DMA,
          pltpu.SemaphoreType.DMA,
      ],
      compiler_params=pltpu.CompilerParams(use_tc_tiling_on_sc=False),
  )(x, idxt, da)


def kernel(x, idx, dis, angle_t):
  idxt = idx.astype(jnp.int32).T
  da = jnp.concatenate([dis, angle_t], axis=1)
  return _run(x, idxt, da)
